# Initial kernel scaffold; baseline (speedup 1.0000x reference)
import jax
import jax.numpy as jnp
from jax.experimental import pallas as pl


def kernel(anchor, feature):
    raise NotImplementedError("write your pallas kernel here")


# speedup vs baseline: 142.6067x; 142.6067x over previous
import jax
import jax.numpy as jnp
from jax.experimental import pallas as pl
from jax.experimental.pallas import tpu as pltpu

_ATOMS = 51
_V_MIN = -1.0
_V_MAX = 1.0
_BLOCK_ROWS = 2048


def _projection_matrix() -> jnp.ndarray:
    atoms = _ATOMS
    delta = (_V_MAX - _V_MIN) / (atoms - 1)
    supports = jnp.linspace(_V_MIN, _V_MAX, atoms).astype(jnp.float32)
    tz = jnp.clip(supports, _V_MIN, _V_MAX)
    b = (tz - _V_MIN) / delta
    l = jnp.floor(b).astype(jnp.int32)
    u = jnp.ceil(b).astype(jnp.int32)
    l = jnp.where((u > 0) & (l == u), l - 1, l)
    u = jnp.where((l < atoms - 1) & (l == u), u + 1, u)
    w_l = u.astype(jnp.float32) - b
    w_u = b - l.astype(jnp.float32)
    rows = jnp.arange(atoms)
    p = jnp.zeros((atoms, atoms), jnp.float32)
    p = p.at[rows, l].add(w_l)
    p = p.at[rows, u].add(w_u)
    return p


def _loss_kernel(p_ref, a_ref, f_ref, out_ref):
    logf = jnp.log(f_ref[...] + 1e-16)
    skewed = jax.lax.dot_general(
        a_ref[...], p_ref[...],
        dimension_numbers=(((1,), (0,)), ((), ())),
        preferred_element_type=jnp.float32,
        precision=jax.lax.Precision.HIGHEST,
    )
    partial = jnp.sum(skewed * logf)

    @pl.when(pl.program_id(0) == 0)
    def _init():
        out_ref[0, 0] = 0.0

    out_ref[0, 0] += partial


def kernel(anchor, feature):
    batch = anchor.shape[0]
    grid = batch // _BLOCK_ROWS
    p = _projection_matrix()
    acc = pl.pallas_call(
        _loss_kernel,
        grid=(grid,),
        in_specs=[
            pl.BlockSpec((_ATOMS, _ATOMS), lambda i: (0, 0)),
            pl.BlockSpec((_BLOCK_ROWS, _ATOMS), lambda i: (i, 0)),
            pl.BlockSpec((_BLOCK_ROWS, _ATOMS), lambda i: (i, 0)),
        ],
        out_specs=pl.BlockSpec(memory_space=pltpu.SMEM),
        out_shape=jax.ShapeDtypeStruct((1, 1), jnp.float32),
    )(p, anchor, feature)
    return -(acc[0, 0] / jnp.float32(batch))


# speedup vs baseline: 184.4905x; 1.2937x over previous
import jax
import jax.numpy as jnp
import numpy as np
from jax.experimental import pallas as pl
from jax.experimental.pallas import tpu as pltpu

_ATOMS = 51
_V_MIN = -1.0
_V_MAX = 1.0
_BLOCK_ROWS = 2048


def _projection_matrix() -> np.ndarray:
    atoms = _ATOMS
    delta = (_V_MAX - _V_MIN) / (atoms - 1)
    supports = np.linspace(_V_MIN, _V_MAX, atoms).astype(np.float32)
    tz = np.clip(supports, _V_MIN, _V_MAX).astype(np.float32)
    b = ((tz - np.float32(_V_MIN)) / np.float32(delta)).astype(np.float32)
    l = np.floor(b).astype(np.int32)
    u = np.ceil(b).astype(np.int32)
    l = np.where((u > 0) & (l == u), l - 1, l)
    u = np.where((l < atoms - 1) & (l == u), u + 1, u)
    w_l = u.astype(np.float32) - b
    w_u = b - l.astype(np.float32)
    p = np.zeros((atoms, atoms), np.float32)
    np.add.at(p, (np.arange(atoms), l), w_l)
    np.add.at(p, (np.arange(atoms), u), w_u)
    return p


_P_CONST = _projection_matrix()


def _loss_kernel(p_ref, a_ref, f_ref, out_ref):
    logf = jnp.log(f_ref[...] + 1e-16)
    skewed = jax.lax.dot_general(
        a_ref[...], p_ref[...],
        dimension_numbers=(((1,), (0,)), ((), ())),
        preferred_element_type=jnp.float32,
        precision=jax.lax.Precision.HIGHEST,
    )
    partial = jnp.sum(skewed * logf)

    @pl.when(pl.program_id(0) == 0)
    def _init():
        out_ref[0, 0] = 0.0

    out_ref[0, 0] += partial


def kernel(anchor, feature):
    batch = anchor.shape[0]
    grid = batch // _BLOCK_ROWS
    p = jnp.asarray(_P_CONST)
    acc = pl.pallas_call(
        _loss_kernel,
        grid=(grid,),
        in_specs=[
            pl.BlockSpec((_ATOMS, _ATOMS), lambda i: (0, 0)),
            pl.BlockSpec((_BLOCK_ROWS, _ATOMS), lambda i: (i, 0)),
            pl.BlockSpec((_BLOCK_ROWS, _ATOMS), lambda i: (i, 0)),
        ],
        out_specs=pl.BlockSpec(memory_space=pltpu.SMEM),
        out_shape=jax.ShapeDtypeStruct((1, 1), jnp.float32),
    )(p, anchor, feature)
    return -(acc[0, 0] / jnp.float32(batch))


# speedup vs baseline: 450.1230x; 2.4398x over previous
import jax
import jax.numpy as jnp
import numpy as np
from jax.experimental import pallas as pl
from jax.experimental.pallas import tpu as pltpu

_ATOMS = 51
_V_MIN = -1.0
_V_MAX = 1.0
_BLOCK_COLS = 2048


def _projection_matrix() -> np.ndarray:
    atoms = _ATOMS
    delta = (_V_MAX - _V_MIN) / (atoms - 1)
    supports = np.linspace(_V_MIN, _V_MAX, atoms).astype(np.float32)
    tz = np.clip(supports, _V_MIN, _V_MAX).astype(np.float32)
    b = ((tz - np.float32(_V_MIN)) / np.float32(delta)).astype(np.float32)
    l = np.floor(b).astype(np.int32)
    u = np.ceil(b).astype(np.int32)
    l = np.where((u > 0) & (l == u), l - 1, l)
    u = np.where((l < atoms - 1) & (l == u), u + 1, u)
    w_l = u.astype(np.float32) - b
    w_u = b - l.astype(np.float32)
    p = np.zeros((atoms, atoms), np.float32)
    np.add.at(p, (np.arange(atoms), l), w_l)
    np.add.at(p, (np.arange(atoms), u), w_u)
    return p


_P_CONST = _projection_matrix()


def _loss_kernel(p_ref, a_ref, f_ref, out_ref):
    logf = jnp.log(f_ref[...] + 1e-16)
    skewed = jax.lax.dot_general(
        p_ref[...], a_ref[...],
        dimension_numbers=(((1,), (0,)), ((), ())),
        preferred_element_type=jnp.float32,
        precision=jax.lax.Precision.HIGHEST,
    )
    partial = jnp.sum(skewed * logf)

    @pl.when(pl.program_id(0) == 0)
    def _init():
        out_ref[0, 0] = 0.0

    out_ref[0, 0] += partial


def kernel(anchor, feature):
    batch = anchor.shape[0]
    a_t = anchor.T
    f_t = feature.T
    grid = batch // _BLOCK_COLS
    p_t = jnp.asarray(_P_CONST.T.copy())
    acc = pl.pallas_call(
        _loss_kernel,
        grid=(grid,),
        in_specs=[
            pl.BlockSpec((_ATOMS, _ATOMS), lambda i: (0, 0)),
            pl.BlockSpec((_ATOMS, _BLOCK_COLS), lambda i: (0, i)),
            pl.BlockSpec((_ATOMS, _BLOCK_COLS), lambda i: (0, i)),
        ],
        out_specs=pl.BlockSpec(memory_space=pltpu.SMEM),
        out_shape=jax.ShapeDtypeStruct((1, 1), jnp.float32),
    )(p_t, a_t, f_t)
    return -(acc[0, 0] / jnp.float32(batch))


# speedup vs baseline: 528.6478x; 1.1745x over previous
import jax
import jax.numpy as jnp
import numpy as np
from jax.experimental import pallas as pl
from jax.experimental.pallas import tpu as pltpu

_ATOMS = 51
_V_MIN = -1.0
_V_MAX = 1.0
_BLOCK_COLS = 2048


def _projection_matrix() -> np.ndarray:
    atoms = _ATOMS
    delta = (_V_MAX - _V_MIN) / (atoms - 1)
    supports = np.linspace(_V_MIN, _V_MAX, atoms).astype(np.float32)
    tz = np.clip(supports, _V_MIN, _V_MAX).astype(np.float32)
    b = ((tz - np.float32(_V_MIN)) / np.float32(delta)).astype(np.float32)
    l = np.floor(b).astype(np.int32)
    u = np.ceil(b).astype(np.int32)
    l = np.where((u > 0) & (l == u), l - 1, l)
    u = np.where((l < atoms - 1) & (l == u), u + 1, u)
    w_l = u.astype(np.float32) - b
    w_u = b - l.astype(np.float32)
    p = np.zeros((atoms, atoms), np.float32)
    np.add.at(p, (np.arange(atoms), l), w_l)
    np.add.at(p, (np.arange(atoms), u), w_u)
    return p


_P_CONST = _projection_matrix()


def _loss_kernel(p_ref, a_ref, f_ref, out_ref):
    logf = jnp.log(f_ref[...] + 1e-16)
    skewed = jax.lax.dot_general(
        p_ref[...], a_ref[...],
        dimension_numbers=(((1,), (0,)), ((), ())),
        preferred_element_type=jnp.float32,
        precision=jax.lax.Precision.DEFAULT,
    )
    partial = jnp.sum(skewed * logf)

    @pl.when(pl.program_id(0) == 0)
    def _init():
        out_ref[0, 0] = 0.0

    out_ref[0, 0] += partial


def kernel(anchor, feature):
    batch = anchor.shape[0]
    a_t = anchor.T
    f_t = feature.T
    grid = batch // _BLOCK_COLS
    p_t = jnp.asarray(_P_CONST.T.copy())
    acc = pl.pallas_call(
        _loss_kernel,
        grid=(grid,),
        in_specs=[
            pl.BlockSpec((_ATOMS, _ATOMS), lambda i: (0, 0)),
            pl.BlockSpec((_ATOMS, _BLOCK_COLS), lambda i: (0, i)),
            pl.BlockSpec((_ATOMS, _BLOCK_COLS), lambda i: (0, i)),
        ],
        out_specs=pl.BlockSpec(memory_space=pltpu.SMEM),
        out_shape=jax.ShapeDtypeStruct((1, 1), jnp.float32),
    )(p_t, a_t, f_t)
    return -(acc[0, 0] / jnp.float32(batch))


# speedup vs baseline: 670.7918x; 1.2689x over previous
import jax
import jax.numpy as jnp
import numpy as np
from jax.experimental import pallas as pl
from jax.experimental.pallas import tpu as pltpu

_ATOMS = 51
_V_MIN = -1.0
_V_MAX = 1.0
_BLOCK_COLS = 4096


def _projection_matrix() -> np.ndarray:
    atoms = _ATOMS
    delta = (_V_MAX - _V_MIN) / (atoms - 1)
    supports = np.linspace(_V_MIN, _V_MAX, atoms).astype(np.float32)
    tz = np.clip(supports, _V_MIN, _V_MAX).astype(np.float32)
    b = ((tz - np.float32(_V_MIN)) / np.float32(delta)).astype(np.float32)
    l = np.floor(b).astype(np.int32)
    u = np.ceil(b).astype(np.int32)
    l = np.where((u > 0) & (l == u), l - 1, l)
    u = np.where((l < atoms - 1) & (l == u), u + 1, u)
    w_l = u.astype(np.float32) - b
    w_u = b - l.astype(np.float32)
    p = np.zeros((atoms, atoms), np.float32)
    np.add.at(p, (np.arange(atoms), l), w_l)
    np.add.at(p, (np.arange(atoms), u), w_u)
    return p


_P_CONST = _projection_matrix()


def _loss_kernel(p_ref, a_ref, f_ref, out_ref):
    logf = jnp.log(f_ref[...] + 1e-16)
    skewed = jax.lax.dot_general(
        p_ref[...], a_ref[...],
        dimension_numbers=(((1,), (0,)), ((), ())),
        preferred_element_type=jnp.float32,
        precision=jax.lax.Precision.DEFAULT,
    )
    partial = jnp.sum(skewed * logf)

    @pl.when(pl.program_id(0) == 0)
    def _init():
        out_ref[0, 0] = 0.0

    out_ref[0, 0] += partial


def kernel(anchor, feature):
    batch = anchor.shape[0]
    a_t = anchor.T
    f_t = feature.T
    grid = batch // _BLOCK_COLS
    p_t = jnp.asarray(_P_CONST.T.copy())
    acc = pl.pallas_call(
        _loss_kernel,
        grid=(grid,),
        in_specs=[
            pl.BlockSpec((_ATOMS, _ATOMS), lambda i: (0, 0)),
            pl.BlockSpec((_ATOMS, _BLOCK_COLS), lambda i: (0, i)),
            pl.BlockSpec((_ATOMS, _BLOCK_COLS), lambda i: (0, i)),
        ],
        out_specs=pl.BlockSpec(memory_space=pltpu.SMEM),
        out_shape=jax.ShapeDtypeStruct((1, 1), jnp.float32),
    )(p_t, a_t, f_t)
    return -(acc[0, 0] / jnp.float32(batch))


# speedup vs baseline: 710.0887x; 1.0586x over previous
import jax
import jax.numpy as jnp
import numpy as np
from jax.experimental import pallas as pl
from jax.experimental.pallas import tpu as pltpu

_ATOMS = 51
_V_MIN = -1.0
_V_MAX = 1.0
_BLOCK_COLS = 8192


def _projection_matrix() -> np.ndarray:
    atoms = _ATOMS
    delta = (_V_MAX - _V_MIN) / (atoms - 1)
    supports = np.linspace(_V_MIN, _V_MAX, atoms).astype(np.float32)
    tz = np.clip(supports, _V_MIN, _V_MAX).astype(np.float32)
    b = ((tz - np.float32(_V_MIN)) / np.float32(delta)).astype(np.float32)
    l = np.floor(b).astype(np.int32)
    u = np.ceil(b).astype(np.int32)
    l = np.where((u > 0) & (l == u), l - 1, l)
    u = np.where((l < atoms - 1) & (l == u), u + 1, u)
    w_l = u.astype(np.float32) - b
    w_u = b - l.astype(np.float32)
    p = np.zeros((atoms, atoms), np.float32)
    np.add.at(p, (np.arange(atoms), l), w_l)
    np.add.at(p, (np.arange(atoms), u), w_u)
    return p


_P_CONST = _projection_matrix()


def _loss_kernel(p_ref, a_ref, f_ref, out_ref):
    logf = jnp.log(f_ref[...] + 1e-16)
    skewed = jax.lax.dot_general(
        p_ref[...], a_ref[...],
        dimension_numbers=(((1,), (0,)), ((), ())),
        preferred_element_type=jnp.float32,
        precision=jax.lax.Precision.DEFAULT,
    )
    partial = jnp.sum(skewed * logf)

    @pl.when(pl.program_id(0) == 0)
    def _init():
        out_ref[0, 0] = 0.0

    out_ref[0, 0] += partial


def kernel(anchor, feature):
    batch = anchor.shape[0]
    a_t = anchor.T
    f_t = feature.T
    grid = batch // _BLOCK_COLS
    p_t = jnp.asarray(_P_CONST.T.copy())
    acc = pl.pallas_call(
        _loss_kernel,
        grid=(grid,),
        in_specs=[
            pl.BlockSpec((_ATOMS, _ATOMS), lambda i: (0, 0)),
            pl.BlockSpec((_ATOMS, _BLOCK_COLS), lambda i: (0, i)),
            pl.BlockSpec((_ATOMS, _BLOCK_COLS), lambda i: (0, i)),
        ],
        out_specs=pl.BlockSpec(memory_space=pltpu.SMEM),
        out_shape=jax.ShapeDtypeStruct((1, 1), jnp.float32),
    )(p_t, a_t, f_t)
    return -(acc[0, 0] / jnp.float32(batch))


# speedup vs baseline: 739.0790x; 1.0408x over previous
import jax
import jax.numpy as jnp
import numpy as np
from jax.experimental import pallas as pl
from jax.experimental.pallas import tpu as pltpu

_ATOMS = 51
_V_MIN = -1.0
_V_MAX = 1.0
_BLOCK_COLS = 16384


def _projection_matrix() -> np.ndarray:
    atoms = _ATOMS
    delta = (_V_MAX - _V_MIN) / (atoms - 1)
    supports = np.linspace(_V_MIN, _V_MAX, atoms).astype(np.float32)
    tz = np.clip(supports, _V_MIN, _V_MAX).astype(np.float32)
    b = ((tz - np.float32(_V_MIN)) / np.float32(delta)).astype(np.float32)
    l = np.floor(b).astype(np.int32)
    u = np.ceil(b).astype(np.int32)
    l = np.where((u > 0) & (l == u), l - 1, l)
    u = np.where((l < atoms - 1) & (l == u), u + 1, u)
    w_l = u.astype(np.float32) - b
    w_u = b - l.astype(np.float32)
    p = np.zeros((atoms, atoms), np.float32)
    np.add.at(p, (np.arange(atoms), l), w_l)
    np.add.at(p, (np.arange(atoms), u), w_u)
    return p


_P_CONST = _projection_matrix()


def _loss_kernel(p_ref, a_ref, f_ref, out_ref):
    logf = jnp.log(f_ref[...] + 1e-16)
    skewed = jax.lax.dot_general(
        p_ref[...], a_ref[...],
        dimension_numbers=(((1,), (0,)), ((), ())),
        preferred_element_type=jnp.float32,
        precision=jax.lax.Precision.DEFAULT,
    )
    partial = jnp.sum(skewed * logf)

    @pl.when(pl.program_id(0) == 0)
    def _init():
        out_ref[0, 0] = 0.0

    out_ref[0, 0] += partial


def kernel(anchor, feature):
    batch = anchor.shape[0]
    a_t = anchor.T
    f_t = feature.T
    grid = batch // _BLOCK_COLS
    p_t = jnp.asarray(_P_CONST.T.copy())
    acc = pl.pallas_call(
        _loss_kernel,
        grid=(grid,),
        in_specs=[
            pl.BlockSpec((_ATOMS, _ATOMS), lambda i: (0, 0)),
            pl.BlockSpec((_ATOMS, _BLOCK_COLS), lambda i: (0, i)),
            pl.BlockSpec((_ATOMS, _BLOCK_COLS), lambda i: (0, i)),
        ],
        out_specs=pl.BlockSpec(memory_space=pltpu.SMEM),
        out_shape=jax.ShapeDtypeStruct((1, 1), jnp.float32),
    )(p_t, a_t, f_t)
    return -(acc[0, 0] / jnp.float32(batch))


# speedup vs baseline: 746.5593x; 1.0101x over previous
import jax
import jax.numpy as jnp
import numpy as np
from jax.experimental import pallas as pl
from jax.experimental.pallas import tpu as pltpu

_ATOMS = 51
_V_MIN = -1.0
_V_MAX = 1.0


def _projection_matrix() -> np.ndarray:
    atoms = _ATOMS
    delta = (_V_MAX - _V_MIN) / (atoms - 1)
    supports = np.linspace(_V_MIN, _V_MAX, atoms).astype(np.float32)
    tz = np.clip(supports, _V_MIN, _V_MAX).astype(np.float32)
    b = ((tz - np.float32(_V_MIN)) / np.float32(delta)).astype(np.float32)
    l = np.floor(b).astype(np.int32)
    u = np.ceil(b).astype(np.int32)
    l = np.where((u > 0) & (l == u), l - 1, l)
    u = np.where((l < atoms - 1) & (l == u), u + 1, u)
    w_l = u.astype(np.float32) - b
    w_u = b - l.astype(np.float32)
    p = np.zeros((atoms, atoms), np.float32)
    np.add.at(p, (np.arange(atoms), l), w_l)
    np.add.at(p, (np.arange(atoms), u), w_u)
    return p


_P_CONST = _projection_matrix()


def _loss_kernel(p_ref, a_ref, f_ref, out_ref):
    logf = jnp.log(f_ref[...] + 1e-16)
    skewed = jax.lax.dot_general(
        p_ref[...], a_ref[...],
        dimension_numbers=(((1,), (0,)), ((), ())),
        preferred_element_type=jnp.float32,
        precision=jax.lax.Precision.DEFAULT,
    )
    out_ref[0, 0] = jnp.sum(skewed * logf)


def kernel(anchor, feature):
    batch = anchor.shape[0]
    a_t = anchor.T
    f_t = feature.T
    p_t = jnp.asarray(_P_CONST.T.copy())
    acc = pl.pallas_call(
        _loss_kernel,
        in_specs=[
            pl.BlockSpec(memory_space=pltpu.VMEM),
            pl.BlockSpec(memory_space=pltpu.VMEM),
            pl.BlockSpec(memory_space=pltpu.VMEM),
        ],
        out_specs=pl.BlockSpec(memory_space=pltpu.SMEM),
        out_shape=jax.ShapeDtypeStruct((1, 1), jnp.float32),
    )(p_t, a_t, f_t)
    return -(acc[0, 0] / jnp.float32(batch))
